# baseline (speedup 1.0000x reference)
import functools
import math

import jax
import jax.numpy as jnp
from jax import lax
from jax.experimental import pallas as pl
from jax.experimental.pallas import tpu as pltpu
from jax.experimental.pallas import tpu_sc as plsc

VOCAB = 1000000
EMB = 64
PAD = 128
SCALE = math.sqrt(EMB)

NUM_WORKERS = 32
BDIM = 4096
TDIM = 200
B_TOTAL = BDIM * TDIM
LANES = 16

_MESH = lambda: plsc.VectorSubcoreMesh(core_axis_name="c", subcore_axis_name="s")


def _wid():
  return lax.axis_index("s") * 2 + lax.axis_index("c")


KA_CHUNK = 160
KA_NCHUNK = VOCAB // KA_CHUNK


def _make_ka():
  @functools.partial(
      pl.kernel,
      mesh=_MESH(),
      out_type=jax.ShapeDtypeStruct((VOCAB // 2, PAD), jnp.float32),
      scratch_types=[pltpu.VMEM((KA_CHUNK, EMB), jnp.float32),
                     pltpu.VMEM((KA_CHUNK, EMB), jnp.float32),
                     pltpu.VMEM((KA_CHUNK // 2, PAD), jnp.float32),
                     pltpu.VMEM((KA_CHUNK // 2, PAD), jnp.float32),
                     pltpu.SemaphoreType.DMA, pltpu.SemaphoreType.DMA,
                     pltpu.SemaphoreType.DMA, pltpu.SemaphoreType.DMA],
  )
  def ka(table_hbm, out_hbm, ib0, ib1, ob0, ob1, is0, is1, os0, os1):
    w = _wid()
    nchunk = (KA_NCHUNK - w + NUM_WORKERS - 1) // NUM_WORKERS
    ibufs = (ib0, ib1)
    obufs = (ob0, ob1)
    isems = (is0, is1)
    osems = (os0, os1)

    def do_chunk(c, ib, ob, isem, osem, first):
      r0 = pl.multiple_of(c * KA_CHUNK, 8)
      pltpu.async_copy(table_hbm.at[pl.ds(r0, KA_CHUNK)], ib, isem).wait()

      def pair_body(j, carry):
        for h in range(2):
          for i in range(EMB // LANES):
            ob[j, pl.ds(h * EMB + i * LANES, LANES)] = (
                ib[2 * j + h, pl.ds(i * LANES, LANES)] * SCALE)
        return carry
      lax.fori_loop(0, KA_CHUNK // 2, pair_body, 0, unroll=2)

      @pl.when(jnp.logical_not(first))
      def _():
        pltpu.make_async_copy(
            ob, out_hbm.at[pl.ds(0, KA_CHUNK // 2)], osem).wait()
      pltpu.async_copy(
          ob, out_hbm.at[pl.ds(pl.multiple_of(r0 // 2, 8), KA_CHUNK // 2)],
          osem)

    def body(i, carry):
      c = w + i * NUM_WORKERS
      for par in range(2):
        @pl.when(lax.rem(i, 2) == par)
        def _(par=par):
          do_chunk(c, ibufs[par], obufs[par], isems[par], osems[par], i < 2)
      return carry

    lax.fori_loop(0, nchunk, body, 0)
    for par in range(2):
      pltpu.make_async_copy(
          obufs[par], out_hbm.at[pl.ds(0, KA_CHUNK // 2)], osems[par]).wait()

  return ka


KB_PER_W = B_TOTAL // NUM_WORKERS
KB_CHUNK = 128
KB_NCHUNK = KB_PER_W // KB_CHUNK
KB_NBUF = 2
KB_GROUP = 2 * KB_NBUF
KB_NBODY = KB_NCHUNK // KB_GROUP


def _make_kb():
  rows_scratch = [pltpu.VMEM((KB_CHUNK, EMB), jnp.float32)
                  for _ in range(2 * KB_NBUF)]
  gsem_scratch = [pltpu.SemaphoreType.DMA for _ in range(2 * KB_NBUF)]

  @functools.partial(
      pl.kernel,
      mesh=_MESH(),
      out_type=jax.ShapeDtypeStruct((B_TOTAL, EMB), jnp.float32),
      compiler_params=pltpu.CompilerParams(use_tc_tiling_on_sc=False),
      scratch_types=[pltpu.VMEM((KB_PER_W,), jnp.int32)]
      + rows_scratch
      + gsem_scratch
      + [pltpu.SemaphoreType.DMA, pltpu.SemaphoreType.DMA],
  )
  def kb(tokens_hbm, table_hbm, out_hbm, idx_v, *scratch):
    rows = scratch[:2 * KB_NBUF]
    gsem = scratch[2 * KB_NBUF:4 * KB_NBUF]
    osem = scratch[4 * KB_NBUF:]
    rows_ab = (rows[:KB_NBUF], rows[KB_NBUF:])
    gsem_ab = (gsem[:KB_NBUF], gsem[KB_NBUF:])

    base = _wid() * KB_PER_W
    pltpu.sync_copy(tokens_hbm.at[pl.ds(base, KB_PER_W)], idx_v)

    def body(g, carry):
      goff = g * KB_GROUP * KB_CHUNK
      handles = [None] * 2
      for s in range(2):
        @pl.when(g > 0)
        def _(s=s):
          for b in range(KB_NBUF):
            pltpu.make_async_copy(
                rows_ab[s][b], out_hbm.at[pl.ds(0, KB_CHUNK)],
                osem[s]).wait()
        handles[s] = [
            pltpu.async_copy(
                table_hbm.at[idx_v.at[pl.ds(goff + (s * KB_NBUF + b)
                                            * KB_CHUNK, KB_CHUNK)]],
                rows_ab[s][b], gsem_ab[s][b])
            for b in range(KB_NBUF)
        ]
      for s in range(2):
        for b in range(KB_NBUF):
          handles[s][b].wait()
          pltpu.async_copy(
              rows_ab[s][b],
              out_hbm.at[pl.ds(base + goff + (s * KB_NBUF + b) * KB_CHUNK,
                               KB_CHUNK)],
              osem[s])
      return carry

    lax.fori_loop(0, KB_NBODY, body, 0)
    for s in range(2):
      for b in range(KB_NBUF):
        pltpu.make_async_copy(
            rows_ab[s][b], out_hbm.at[pl.ds(0, KB_CHUNK)], osem[s]).wait()

  return kb


KC_PER_W = (B_TOTAL // 2) // NUM_WORKERS
KC_CHUNK = 40
KC_OUT = 2 * KC_CHUNK
KC_SEG = 40
KC_NCHUNK = KC_PER_W // KC_CHUNK


def _make_kc():
  @functools.partial(
      pl.kernel,
      mesh=_MESH(),
      out_type=jax.ShapeDtypeStruct((BDIM, TDIM, EMB), jnp.float32),
      scratch_types=[pltpu.VMEM((KC_CHUNK, PAD), jnp.float32),
                     pltpu.VMEM((KC_CHUNK, PAD), jnp.float32),
                     pltpu.VMEM((KC_OUT, EMB), jnp.float32),
                     pltpu.VMEM((KC_OUT, EMB), jnp.float32),
                     pltpu.SemaphoreType.DMA, pltpu.SemaphoreType.DMA,
                     pltpu.SemaphoreType.DMA, pltpu.SemaphoreType.DMA],
  )
  def kc(rows_hbm, out_hbm, ib0, ib1, ob0, ob1, is0, is1, os0, os1):
    w = _wid()
    base = w * KC_PER_W
    ibufs = (ib0, ib1)
    obufs = (ob0, ob1)
    isems = (is0, is1)
    osems = (os0, os1)

    def do_chunk(c, ib, ob, isem, osem, first):
      r0 = pl.multiple_of(base + c * KC_CHUNK, 8)
      pltpu.async_copy(rows_hbm.at[pl.ds(r0, KC_CHUNK)], ib, isem).wait()

      def pair_body(j, carry):
        for h in range(2):
          for i in range(EMB // LANES):
            ob[2 * j + h, pl.ds(i * LANES, LANES)] = (
                ib[j, pl.ds(h * EMB + i * LANES, LANES)])
        return carry
      lax.fori_loop(0, KC_CHUNK, pair_body, 0, unroll=2)

      @pl.when(jnp.logical_not(first))
      def _():
        for seg in range(KC_OUT // KC_SEG):
          pltpu.make_async_copy(
              ob.at[pl.ds(seg * KC_SEG, KC_SEG)],
              out_hbm.at[0, pl.ds(0, KC_SEG)], osem).wait()
      for seg in range(KC_OUT // KC_SEG):
        orow = 2 * r0 + seg * KC_SEG
        brow = orow // TDIM
        t0 = pl.multiple_of(orow - brow * TDIM, 8)
        pltpu.async_copy(
            ob.at[pl.ds(seg * KC_SEG, KC_SEG)],
            out_hbm.at[brow, pl.ds(t0, KC_SEG)], osem)

    def body(c, carry):
      for par in range(2):
        @pl.when(lax.rem(c, 2) == par)
        def _(par=par):
          do_chunk(c, ibufs[par], obufs[par], isems[par], osems[par], c < 2)
      return carry

    lax.fori_loop(0, KC_NCHUNK, body, 0)
    for par in range(2):
      for seg in range(KC_OUT // KC_SEG):
        pltpu.make_async_copy(
            obufs[par].at[pl.ds(seg * KC_SEG, KC_SEG)],
            out_hbm.at[0, pl.ds(0, KC_SEG)], osems[par]).wait()

  return kc


_ka = _make_ka()
_kb = _make_kb()
_kc = _make_kc()


def kernel(tokens, table):
  flat = tokens.reshape(-1).astype(jnp.int32)
  compact = _ka(table)
  compact64 = compact.reshape(VOCAB, EMB)
  gathered = _kb(flat, compact64)
  packed = gathered.reshape(B_TOTAL // 2, PAD)
  return _kc(packed)


# speedup vs baseline: 1.2083x; 1.2083x over previous
import functools
import math

import jax
import jax.numpy as jnp
from jax import lax
from jax.experimental import pallas as pl
from jax.experimental.pallas import tpu as pltpu
from jax.experimental.pallas import tpu_sc as plsc

VOCAB = 1000000
EMB = 64
PAD = 128
SCALE = math.sqrt(EMB)

NUM_WORKERS = 32
BDIM = 4096
TDIM = 200
B_TOTAL = BDIM * TDIM
LANES = 16

_MESH = lambda: plsc.VectorSubcoreMesh(core_axis_name="c", subcore_axis_name="s")


def _wid():
  return lax.axis_index("s") * 2 + lax.axis_index("c")


KA_B = 1600
KA_GRID = VOCAB // KA_B


def _make_ka_tc():
  def body(tbl_ref, out_ref):
    blk = tbl_ref[...].reshape(KA_B // 2, 2, EMB)
    out_ref[...] = jnp.concatenate(
        [blk[:, 0, :], blk[:, 1, :]], axis=1) * SCALE

  return pl.pallas_call(
      body,
      grid=(KA_GRID,),
      in_specs=[pl.BlockSpec((KA_B, EMB), lambda i: (i, 0))],
      out_specs=pl.BlockSpec((KA_B // 2, PAD), lambda i: (i, 0)),
      out_shape=jax.ShapeDtypeStruct((VOCAB // 2, PAD), jnp.float32),
  )


KC_B = 800
KC_GRID = (B_TOTAL // 2) // KC_B


def _make_kc_tc():
  def body(in_ref, out_ref):
    blk = in_ref[...]
    lo = blk[:, :EMB]
    hi = blk[:, EMB:]
    inter = jnp.stack([lo, hi], axis=1).reshape(2 * KC_B, EMB)
    out_ref[...] = inter.reshape(8, TDIM, EMB)

  return pl.pallas_call(
      body,
      grid=(KC_GRID,),
      in_specs=[pl.BlockSpec((KC_B, PAD), lambda i: (i, 0))],
      out_specs=pl.BlockSpec((8, TDIM, EMB), lambda i: (i, 0, 0)),
      out_shape=jax.ShapeDtypeStruct((BDIM, TDIM, EMB), jnp.float32),
  )


KA_CHUNK = 160
KA_NCHUNK = VOCAB // KA_CHUNK


def _make_ka():
  @functools.partial(
      pl.kernel,
      mesh=_MESH(),
      out_type=jax.ShapeDtypeStruct((VOCAB // 2, PAD), jnp.float32),
      scratch_types=[pltpu.VMEM((KA_CHUNK, EMB), jnp.float32),
                     pltpu.VMEM((KA_CHUNK, EMB), jnp.float32),
                     pltpu.VMEM((KA_CHUNK // 2, PAD), jnp.float32),
                     pltpu.VMEM((KA_CHUNK // 2, PAD), jnp.float32),
                     pltpu.SemaphoreType.DMA, pltpu.SemaphoreType.DMA,
                     pltpu.SemaphoreType.DMA, pltpu.SemaphoreType.DMA],
  )
  def ka(table_hbm, out_hbm, ib0, ib1, ob0, ob1, is0, is1, os0, os1):
    w = _wid()
    nchunk = (KA_NCHUNK - w + NUM_WORKERS - 1) // NUM_WORKERS
    ibufs = (ib0, ib1)
    obufs = (ob0, ob1)
    isems = (is0, is1)
    osems = (os0, os1)

    def do_chunk(c, ib, ob, isem, osem, first):
      r0 = pl.multiple_of(c * KA_CHUNK, 8)
      pltpu.async_copy(table_hbm.at[pl.ds(r0, KA_CHUNK)], ib, isem).wait()

      def pair_body(j, carry):
        for h in range(2):
          for i in range(EMB // LANES):
            ob[j, pl.ds(h * EMB + i * LANES, LANES)] = (
                ib[2 * j + h, pl.ds(i * LANES, LANES)] * SCALE)
        return carry
      lax.fori_loop(0, KA_CHUNK // 2, pair_body, 0, unroll=2)

      @pl.when(jnp.logical_not(first))
      def _():
        pltpu.make_async_copy(
            ob, out_hbm.at[pl.ds(0, KA_CHUNK // 2)], osem).wait()
      pltpu.async_copy(
          ob, out_hbm.at[pl.ds(pl.multiple_of(r0 // 2, 8), KA_CHUNK // 2)],
          osem)

    def body(i, carry):
      c = w + i * NUM_WORKERS
      for par in range(2):
        @pl.when(lax.rem(i, 2) == par)
        def _(par=par):
          do_chunk(c, ibufs[par], obufs[par], isems[par], osems[par], i < 2)
      return carry

    lax.fori_loop(0, nchunk, body, 0)
    for par in range(2):
      pltpu.make_async_copy(
          obufs[par], out_hbm.at[pl.ds(0, KA_CHUNK // 2)], osems[par]).wait()

  return ka


KB_PER_W = B_TOTAL // NUM_WORKERS
KB_CHUNK = 128
KB_NCHUNK = KB_PER_W // KB_CHUNK
KB_NBUF = 2
KB_GROUP = 2 * KB_NBUF
KB_NBODY = KB_NCHUNK // KB_GROUP


def _make_kb():
  rows_scratch = [pltpu.VMEM((KB_CHUNK, EMB), jnp.float32)
                  for _ in range(2 * KB_NBUF)]
  gsem_scratch = [pltpu.SemaphoreType.DMA for _ in range(2 * KB_NBUF)]

  @functools.partial(
      pl.kernel,
      mesh=_MESH(),
      out_type=jax.ShapeDtypeStruct((B_TOTAL, EMB), jnp.float32),
      compiler_params=pltpu.CompilerParams(use_tc_tiling_on_sc=False),
      scratch_types=[pltpu.VMEM((KB_PER_W,), jnp.int32)]
      + rows_scratch
      + gsem_scratch
      + [pltpu.SemaphoreType.DMA, pltpu.SemaphoreType.DMA],
  )
  def kb(tokens_hbm, table_hbm, out_hbm, idx_v, *scratch):
    rows = scratch[:2 * KB_NBUF]
    gsem = scratch[2 * KB_NBUF:4 * KB_NBUF]
    osem = scratch[4 * KB_NBUF:]
    rows_ab = (rows[:KB_NBUF], rows[KB_NBUF:])
    gsem_ab = (gsem[:KB_NBUF], gsem[KB_NBUF:])

    base = _wid() * KB_PER_W
    pltpu.sync_copy(tokens_hbm.at[pl.ds(base, KB_PER_W)], idx_v)

    def body(g, carry):
      goff = g * KB_GROUP * KB_CHUNK
      handles = [None] * 2
      for s in range(2):
        @pl.when(g > 0)
        def _(s=s):
          for b in range(KB_NBUF):
            pltpu.make_async_copy(
                rows_ab[s][b], out_hbm.at[pl.ds(0, KB_CHUNK)],
                osem[s]).wait()
        handles[s] = [
            pltpu.async_copy(
                table_hbm.at[idx_v.at[pl.ds(goff + (s * KB_NBUF + b)
                                            * KB_CHUNK, KB_CHUNK)]],
                rows_ab[s][b], gsem_ab[s][b])
            for b in range(KB_NBUF)
        ]
      for s in range(2):
        for b in range(KB_NBUF):
          handles[s][b].wait()
          pltpu.async_copy(
              rows_ab[s][b],
              out_hbm.at[pl.ds(base + goff + (s * KB_NBUF + b) * KB_CHUNK,
                               KB_CHUNK)],
              osem[s])
      return carry

    lax.fori_loop(0, KB_NBODY, body, 0)
    for s in range(2):
      for b in range(KB_NBUF):
        pltpu.make_async_copy(
            rows_ab[s][b], out_hbm.at[pl.ds(0, KB_CHUNK)], osem[s]).wait()

  return kb


KC_PER_W = (B_TOTAL // 2) // NUM_WORKERS
KC_CHUNK = 40
KC_OUT = 2 * KC_CHUNK
KC_SEG = 40
KC_NCHUNK = KC_PER_W // KC_CHUNK


def _make_kc():
  @functools.partial(
      pl.kernel,
      mesh=_MESH(),
      out_type=jax.ShapeDtypeStruct((BDIM, TDIM, EMB), jnp.float32),
      scratch_types=[pltpu.VMEM((KC_CHUNK, PAD), jnp.float32),
                     pltpu.VMEM((KC_CHUNK, PAD), jnp.float32),
                     pltpu.VMEM((KC_OUT, EMB), jnp.float32),
                     pltpu.VMEM((KC_OUT, EMB), jnp.float32),
                     pltpu.SemaphoreType.DMA, pltpu.SemaphoreType.DMA,
                     pltpu.SemaphoreType.DMA, pltpu.SemaphoreType.DMA],
  )
  def kc(rows_hbm, out_hbm, ib0, ib1, ob0, ob1, is0, is1, os0, os1):
    w = _wid()
    base = w * KC_PER_W
    ibufs = (ib0, ib1)
    obufs = (ob0, ob1)
    isems = (is0, is1)
    osems = (os0, os1)

    def do_chunk(c, ib, ob, isem, osem, first):
      r0 = pl.multiple_of(base + c * KC_CHUNK, 8)
      pltpu.async_copy(rows_hbm.at[pl.ds(r0, KC_CHUNK)], ib, isem).wait()

      def pair_body(j, carry):
        for h in range(2):
          for i in range(EMB // LANES):
            ob[2 * j + h, pl.ds(i * LANES, LANES)] = (
                ib[j, pl.ds(h * EMB + i * LANES, LANES)])
        return carry
      lax.fori_loop(0, KC_CHUNK, pair_body, 0, unroll=2)

      @pl.when(jnp.logical_not(first))
      def _():
        for seg in range(KC_OUT // KC_SEG):
          pltpu.make_async_copy(
              ob.at[pl.ds(seg * KC_SEG, KC_SEG)],
              out_hbm.at[0, pl.ds(0, KC_SEG)], osem).wait()
      for seg in range(KC_OUT // KC_SEG):
        orow = 2 * r0 + seg * KC_SEG
        brow = orow // TDIM
        t0 = pl.multiple_of(orow - brow * TDIM, 8)
        pltpu.async_copy(
            ob.at[pl.ds(seg * KC_SEG, KC_SEG)],
            out_hbm.at[brow, pl.ds(t0, KC_SEG)], osem)

    def body(c, carry):
      for par in range(2):
        @pl.when(lax.rem(c, 2) == par)
        def _(par=par):
          do_chunk(c, ibufs[par], obufs[par], isems[par], osems[par], c < 2)
      return carry

    lax.fori_loop(0, KC_NCHUNK, body, 0)
    for par in range(2):
      for seg in range(KC_OUT // KC_SEG):
        pltpu.make_async_copy(
            obufs[par].at[pl.ds(seg * KC_SEG, KC_SEG)],
            out_hbm.at[0, pl.ds(0, KC_SEG)], osems[par]).wait()

  return kc


_ka = _make_ka_tc()
_kb = _make_kb()
_kc = _make_kc_tc()


def kernel(tokens, table):
  flat = tokens.reshape(-1).astype(jnp.int32)
  compact = _ka(table)
  compact64 = compact.reshape(VOCAB, EMB)
  gathered = _kb(flat, compact64)
  packed = gathered.reshape(B_TOTAL // 2, PAD)
  return _kc(packed)


# speedup vs baseline: 1.9177x; 1.5871x over previous
import functools
import math

import jax
import jax.numpy as jnp
from jax import lax
from jax.experimental import pallas as pl
from jax.experimental.pallas import tpu as pltpu
from jax.experimental.pallas import tpu_sc as plsc

VOCAB = 1000000
EMB = 64
SCALE = math.sqrt(EMB)

NUM_WORKERS = 32
B_TOTAL = 4096 * 200
PER_W = B_TOTAL // NUM_WORKERS
CHUNK = 128
NCHUNK = PER_W // CHUNK
NBUF = 2
GROUP = 2 * NBUF
NBODY = NCHUNK // GROUP
LANES = 16


def _make_kernel():
  mesh = plsc.VectorSubcoreMesh(core_axis_name="c", subcore_axis_name="s")

  rows_scratch = [pltpu.VMEM((CHUNK, EMB), jnp.float32)
                  for _ in range(2 * NBUF)]
  gsem_scratch = [pltpu.SemaphoreType.DMA for _ in range(2 * NBUF)]

  @functools.partial(
      pl.kernel,
      mesh=mesh,
      out_type=jax.ShapeDtypeStruct((B_TOTAL, EMB), jnp.float32),
      compiler_params=pltpu.CompilerParams(use_tc_tiling_on_sc=False),
      scratch_types=[pltpu.VMEM((PER_W,), jnp.int32)]
      + rows_scratch
      + gsem_scratch
      + [pltpu.SemaphoreType.DMA, pltpu.SemaphoreType.DMA],
  )
  def emb_kernel(tokens_hbm, table_hbm, out_hbm, idx_v, *scratch):
    rows = scratch[:2 * NBUF]
    gsem = scratch[2 * NBUF:4 * NBUF]
    osem = scratch[4 * NBUF:]
    rows_ab = (rows[:NBUF], rows[NBUF:])
    gsem_ab = (gsem[:NBUF], gsem[NBUF:])

    wid = lax.axis_index("s") * 2 + lax.axis_index("c")
    base = wid * PER_W
    pltpu.sync_copy(tokens_hbm.at[pl.ds(base, PER_W)], idx_v)

    def scale_rows(buf):
      def scale_body(j, carry):
        for i in range(EMB // LANES):
          sl = pl.ds(i * LANES, LANES)
          buf[j, sl] = buf[j, sl] * SCALE
        return carry
      lax.fori_loop(0, CHUNK, scale_body, 0, unroll=2)

    def body(g, carry):
      goff = g * GROUP * CHUNK
      handles = [None] * 2
      for s in range(2):
        @pl.when(g > 0)
        def _(s=s):
          for b in range(NBUF):
            pltpu.make_async_copy(
                rows_ab[s][b], out_hbm.at[pl.ds(0, CHUNK)], osem[s]).wait()
        handles[s] = [
            pltpu.async_copy(
                table_hbm.at[idx_v.at[pl.ds(goff + (s * NBUF + b) * CHUNK,
                                            CHUNK)]],
                rows_ab[s][b], gsem_ab[s][b])
            for b in range(NBUF)
        ]
      for s in range(2):
        for b in range(NBUF):
          handles[s][b].wait()
          scale_rows(rows_ab[s][b])
          pltpu.async_copy(
              rows_ab[s][b],
              out_hbm.at[pl.ds(base + goff + (s * NBUF + b) * CHUNK, CHUNK)],
              osem[s])
      return carry

    lax.fori_loop(0, NBODY, body, 0)
    for s in range(2):
      for b in range(NBUF):
        pltpu.make_async_copy(
            rows_ab[s][b], out_hbm.at[pl.ds(0, CHUNK)], osem[s]).wait()

  return emb_kernel


_emb_kernel = _make_kernel()


def kernel(tokens, table):
  flat = tokens.reshape(-1).astype(jnp.int32)
  out = _emb_kernel(flat, table)
  return out.reshape(tokens.shape + (EMB,))


# speedup vs baseline: 1.9322x; 1.0076x over previous
import functools
import math

import jax
import jax.numpy as jnp
from jax import lax
from jax.experimental import pallas as pl
from jax.experimental.pallas import tpu as pltpu
from jax.experimental.pallas import tpu_sc as plsc

VOCAB = 1000000
EMB = 64
SCALE = math.sqrt(EMB)

NUM_WORKERS = 32
B_TOTAL = 4096 * 200
PER_W = B_TOTAL // NUM_WORKERS
CHUNK = 128
NCHUNK = PER_W // CHUNK
NBUF = 4
GROUP = 2 * NBUF
NBODY = NCHUNK // GROUP
LANES = 16


def _make_kernel():
  mesh = plsc.VectorSubcoreMesh(core_axis_name="c", subcore_axis_name="s")

  rows_scratch = [pltpu.VMEM((CHUNK, EMB), jnp.float32)
                  for _ in range(2 * NBUF)]
  gsem_scratch = [pltpu.SemaphoreType.DMA for _ in range(2 * NBUF)]

  @functools.partial(
      pl.kernel,
      mesh=mesh,
      out_type=jax.ShapeDtypeStruct((B_TOTAL, EMB), jnp.float32),
      compiler_params=pltpu.CompilerParams(use_tc_tiling_on_sc=False),
      scratch_types=[pltpu.VMEM((PER_W,), jnp.int32)]
      + rows_scratch
      + gsem_scratch
      + [pltpu.SemaphoreType.DMA, pltpu.SemaphoreType.DMA],
  )
  def emb_kernel(tokens_hbm, table_hbm, out_hbm, idx_v, *scratch):
    rows = scratch[:2 * NBUF]
    gsem = scratch[2 * NBUF:4 * NBUF]
    osem = scratch[4 * NBUF:]
    rows_ab = (rows[:NBUF], rows[NBUF:])
    gsem_ab = (gsem[:NBUF], gsem[NBUF:])

    wid = lax.axis_index("s") * 2 + lax.axis_index("c")
    base = wid * PER_W
    pltpu.sync_copy(tokens_hbm.at[pl.ds(base, PER_W)], idx_v)

    def scale_rows(buf):
      def scale_body(j, carry):
        for i in range(EMB // LANES):
          sl = pl.ds(i * LANES, LANES)
          buf[j, sl] = buf[j, sl] * SCALE
        return carry
      lax.fori_loop(0, CHUNK, scale_body, 0, unroll=2)

    def body(g, carry):
      goff = g * GROUP * CHUNK
      handles = [None] * 2
      for s in range(2):
        @pl.when(g > 0)
        def _(s=s):
          for b in range(NBUF):
            pltpu.make_async_copy(
                rows_ab[s][b], out_hbm.at[pl.ds(0, CHUNK)], osem[s]).wait()
        handles[s] = [
            pltpu.async_copy(
                table_hbm.at[idx_v.at[pl.ds(goff + (s * NBUF + b) * CHUNK,
                                            CHUNK)]],
                rows_ab[s][b], gsem_ab[s][b])
            for b in range(NBUF)
        ]
      for s in range(2):
        for b in range(NBUF):
          handles[s][b].wait()
          scale_rows(rows_ab[s][b])
          pltpu.async_copy(
              rows_ab[s][b],
              out_hbm.at[pl.ds(base + goff + (s * NBUF + b) * CHUNK, CHUNK)],
              osem[s])
      return carry

    lax.fori_loop(0, NBODY, body, 0)
    for s in range(2):
      for b in range(NBUF):
        pltpu.make_async_copy(
            rows_ab[s][b], out_hbm.at[pl.ds(0, CHUNK)], osem[s]).wait()

  return emb_kernel


_emb_kernel = _make_kernel()


def kernel(tokens, table):
  flat = tokens.reshape(-1).astype(jnp.int32)
  out = _emb_kernel(flat, table)
  return out.reshape(tokens.shape + (EMB,))
